# baseline (speedup 1.0000x reference)
import functools

import jax
import jax.numpy as jnp
from jax.experimental import pallas as pl
from jax.experimental.pallas import tpu as pltpu


def _round_up(x, m):
    return (x + m - 1) // m * m


def _emit_tile_stats(h, h_ref, psum_ref, psq_ref):
    h_ref[...] = h.astype(jnp.bfloat16)
    psum_ref[...] = jnp.broadcast_to(
        jnp.sum(h, axis=0, keepdims=True), psum_ref.shape)
    psq_ref[...] = jnp.broadcast_to(
        jnp.sum(h * h, axis=0, keepdims=True), psq_ref.shape)


def _sym_walk(i, xw, adj_ref, h_ref, psum_ref, psq_ref, acc_scr, tm):
    @pl.when(i == 0)
    def _():
        acc_scr[0:tm, :] = jnp.dot(
            adj_ref[...], xw[0:tm, :], preferred_element_type=jnp.float32)

    @pl.when(i == 1)
    def _():
        a = adj_ref[...]
        acc_scr[0:tm, :] += jnp.dot(
            a, xw[tm:, :], preferred_element_type=jnp.float32)
        acc_scr[tm:, :] = jax.lax.dot_general(
            a, xw[0:tm, :], (((0,), (0,)), ((), ())),
            preferred_element_type=jnp.float32)
        _emit_tile_stats(acc_scr[0:tm, :], h_ref, psum_ref, psq_ref)

    @pl.when(i == 2)
    def _():
        acc_scr[tm:, :] += jnp.dot(
            adj_ref[...], xw[tm:, :], preferred_element_type=jnp.float32)
        _emit_tile_stats(acc_scr[tm:, :], h_ref, psum_ref, psq_ref)


def _l1_body(xw_ref, adj_ref, h_ref, psum_ref, psq_ref, acc_scr, *, tm):
    i = pl.program_id(0)
    _sym_walk(i, xw_ref[...], adj_ref, h_ref, psum_ref, psq_ref, acc_scr, tm)


def _bn_finalize(ps, pq, gamma, beta, inv_n):
    total = jnp.sum(ps, axis=0, keepdims=True) * 0.125
    total_sq = jnp.sum(pq, axis=0, keepdims=True) * 0.125
    mean = total * inv_n
    var = jnp.maximum(total_sq * inv_n - mean * mean, 0.0)
    inv_std = jax.lax.rsqrt(var + 1e-5)
    scale = gamma * inv_std
    shift = beta - mean * scale
    return scale, shift


def _l2_body(h1_ref, ps_ref, pq_ref, g_ref, b_ref, w_ref, adj_ref,
             h_ref, psum_ref, psq_ref, xw_scr, acc_scr, *, tm, inv_n):
    i = pl.program_id(0)

    @pl.when(i == 0)
    def _():
        scale, shift = _bn_finalize(ps_ref[...], pq_ref[...], g_ref[...],
                                    b_ref[...], inv_n)
        a1 = jnp.maximum(
            h1_ref[...].astype(jnp.float32) * scale + shift, 0.0)
        xw_scr[...] = jnp.dot(
            a1.astype(jnp.bfloat16), w_ref[...].astype(jnp.bfloat16),
            preferred_element_type=jnp.float32).astype(jnp.bfloat16)

    _sym_walk(i, xw_scr[...], adj_ref, h_ref, psum_ref, psq_ref, acc_scr, tm)


def _bn_out_body(h_ref, ps_ref, pq_ref, g_ref, b_ref, out_ref, *, inv_n):
    scale, shift = _bn_finalize(ps_ref[...], pq_ref[...], g_ref[...],
                                b_ref[...], inv_n)
    y = h_ref[...].astype(jnp.float32) * scale + shift
    out_ref[...] = jnp.maximum(y, 0.0)


def _adj_index(i):
    return (jnp.maximum(i - 1, 0), jnp.minimum(i, 1))


def _row_index(i):
    return (jnp.maximum(i - 1, 0), 0)


def _propagate1(xw1, adj_pad):
    n_pad = adj_pad.shape[0]
    f_pad = xw1.shape[1]
    tm = n_pad // 2
    body = functools.partial(_l1_body, tm=tm)
    return pl.pallas_call(
        body,
        out_shape=(
            jax.ShapeDtypeStruct((n_pad, f_pad), jnp.bfloat16),
            jax.ShapeDtypeStruct((16, f_pad), jnp.float32),
            jax.ShapeDtypeStruct((16, f_pad), jnp.float32),
        ),
        grid=(3,),
        in_specs=[
            pl.BlockSpec((n_pad, f_pad), lambda i: (0, 0)),
            pl.BlockSpec((tm, tm), _adj_index),
        ],
        out_specs=(
            pl.BlockSpec((tm, f_pad), _row_index),
            pl.BlockSpec((8, f_pad), _row_index),
            pl.BlockSpec((8, f_pad), _row_index),
        ),
        scratch_shapes=[pltpu.VMEM((n_pad, f_pad), jnp.float32)],
        compiler_params=pltpu.CompilerParams(
            dimension_semantics=("arbitrary",),
            vmem_limit_bytes=48 * 1024 * 1024),
    )(xw1, adj_pad)


def _propagate2(h1, ps1, pq1, g1, b1, w2p, adj_pad, n_real):
    n_pad = adj_pad.shape[0]
    f_in = h1.shape[1]
    f_pad = w2p.shape[1]
    tm = n_pad // 2
    body = functools.partial(_l2_body, tm=tm, inv_n=1.0 / n_real)
    return pl.pallas_call(
        body,
        out_shape=(
            jax.ShapeDtypeStruct((n_pad, f_pad), jnp.bfloat16),
            jax.ShapeDtypeStruct((16, f_pad), jnp.float32),
            jax.ShapeDtypeStruct((16, f_pad), jnp.float32),
        ),
        grid=(3,),
        in_specs=[
            pl.BlockSpec((n_pad, f_in), lambda i: (0, 0)),
            pl.BlockSpec(ps1.shape, lambda i: (0, 0)),
            pl.BlockSpec(pq1.shape, lambda i: (0, 0)),
            pl.BlockSpec((1, f_in), lambda i: (0, 0)),
            pl.BlockSpec((1, f_in), lambda i: (0, 0)),
            pl.BlockSpec((f_in, f_pad), lambda i: (0, 0)),
            pl.BlockSpec((tm, tm), _adj_index),
        ],
        out_specs=(
            pl.BlockSpec((tm, f_pad), _row_index),
            pl.BlockSpec((8, f_pad), _row_index),
            pl.BlockSpec((8, f_pad), _row_index),
        ),
        scratch_shapes=[pltpu.VMEM((n_pad, f_pad), jnp.bfloat16),
                        pltpu.VMEM((n_pad, f_pad), jnp.float32)],
        compiler_params=pltpu.CompilerParams(
            dimension_semantics=("arbitrary",),
            vmem_limit_bytes=48 * 1024 * 1024),
    )(h1, ps1, pq1, g1, b1, w2p, adj_pad)


def _bn_out(h2, ps2, pq2, g2, b2, n_real):
    n_pad, f_pad = h2.shape
    tm = n_pad // 2 if n_pad % 2 == 0 and n_pad >= 256 else n_pad
    m_tiles = n_pad // tm
    body = functools.partial(_bn_out_body, inv_n=1.0 / n_real)
    return pl.pallas_call(
        body,
        out_shape=jax.ShapeDtypeStruct((n_pad, f_pad), jnp.float32),
        grid=(m_tiles,),
        in_specs=[
            pl.BlockSpec((tm, f_pad), lambda i: (i, 0)),
            pl.BlockSpec(ps2.shape, lambda i: (0, 0)),
            pl.BlockSpec(pq2.shape, lambda i: (0, 0)),
            pl.BlockSpec((1, f_pad), lambda i: (0, 0)),
            pl.BlockSpec((1, f_pad), lambda i: (0, 0)),
        ],
        out_specs=pl.BlockSpec((tm, f_pad), lambda i: (i, 0)),
        compiler_params=pltpu.CompilerParams(
            dimension_semantics=("parallel",),
            vmem_limit_bytes=32 * 1024 * 1024),
    )(h2, ps2, pq2, g2, b2)


@functools.partial(jax.jit, static_argnames=("num_nodes",))
def _forward(w1, gamma1, beta1, w2, gamma2, beta2, x, adj_pad, num_nodes):
    n = num_nodes
    n_pad = adj_pad.shape[0]
    in_dim = x.shape[1]
    h_dim = w1.shape[1]
    out_dim = w2.shape[1]
    f1_pad = _round_up(h_dim, 128)
    f2_pad = _round_up(out_dim, 128)

    def pad_cols(v, f_pad):
        if v.shape[-1] == f_pad:
            return v.reshape(1, f_pad)
        return jnp.zeros((1, f_pad), jnp.float32).at[:, :v.shape[-1]].set(
            v.reshape(1, -1))

    x_pad = x
    if n_pad != n:
        x_pad = jnp.zeros((n_pad, in_dim), x.dtype).at[:n].set(x)

    w1p = w1
    if h_dim != f1_pad:
        w1p = jnp.zeros((in_dim, f1_pad), jnp.float32).at[:, :h_dim].set(w1)
    w2p = w2
    if h_dim != f1_pad or out_dim != f2_pad:
        w2p = jnp.zeros((f1_pad, f2_pad), jnp.float32)
        w2p = w2p.at[:h_dim, :out_dim].set(w2)

    xw1 = jnp.dot(x_pad.astype(jnp.bfloat16), w1p.astype(jnp.bfloat16),
                  preferred_element_type=jnp.float32).astype(jnp.bfloat16)
    h1, ps1, pq1 = _propagate1(xw1, adj_pad)
    h2, ps2, pq2 = _propagate2(
        h1, ps1, pq1, pad_cols(gamma1, f1_pad), pad_cols(beta1, f1_pad),
        w2p, adj_pad, n)
    out = _bn_out(h2, ps2, pq2, pad_cols(gamma2, f2_pad),
                  pad_cols(beta2, f2_pad), n)
    if n_pad != n or f2_pad != out_dim:
        out = out[:n, :out_dim]
    return out


def kernel(w1, b1, gamma1, beta1, w2, b2, gamma2, beta2, x, adj_pad):
    return _forward(w1, gamma1, beta1, w2, gamma2, beta2, x, adj_pad,
                    num_nodes=x.shape[0])
